# baseline (speedup 1.0000x reference)
import functools

import jax
import jax.numpy as jnp
from jax import lax
from jax.experimental import pallas as pl
from jax.experimental.pallas import tpu as pltpu
from jax.experimental.pallas import tpu_sc as plsc

N = 100000
E = 3200000
D_IN = 58
H = 16
L = 10

NC = 2
NS = 16
NW = NC * NS
B = 128
SB = 256
GB = 4
GE = SB * GB


def _geom(n, e):
    g = {}
    g["EPAD"] = ((e + 2 * NW * GE - 1) // (2 * NW * GE)) * (2 * NW * GE)
    g["EW"] = g["EPAD"] // NW
    g["NG"] = g["EW"] // GE
    assert g["NG"] % 2 == 0 and g["NG"] >= 2
    g["ACC_N"] = n + B
    g["RS"] = g["ACC_N"] // NS
    assert g["RS"] * NS == g["ACC_N"]
    for zch in (42, 14, 2, 1):
        if g["RS"] % zch == 0:
            g["ZCH"] = zch
            g["CH"] = g["RS"] // zch
            break
    g["EROWS"] = g["EPAD"] // SB
    g["WROWS"] = g["EW"] // SB
    return g


_mesh = plsc.VectorSubcoreMesh(core_axis_name="c", subcore_axis_name="s")
_sc_params = pltpu.CompilerParams(use_tc_tiling_on_sc=False)


def _worker_id():
    c = lax.axis_index("c")
    s = lax.axis_index("s")
    return c * NS + s, c, s


def make_deg_kernel(g):
    CH, ZCH, RS, ACC_N, NG, EW = (
        g["CH"], g["ZCH"], g["RS"], g["ACC_N"], g["NG"], g["EW"])

    def body(dst_hbm, zeros_hbm, ones_hbm, out_hbm, zbuf, didx, ones, acc, sem):
        w, c, s = _worker_id()
        pltpu.sync_copy(ones_hbm, ones)
        pltpu.sync_copy(zeros_hbm, zbuf)
        base = s * RS

        def zloop(j, carry):
            pltpu.sync_copy(zbuf, acc.at[pl.ds(base + j * CH, CH)])
            return carry

        lax.fori_loop(0, ZCH, zloop, 0)
        plsc.subcore_barrier()
        rbase = w * (EW // SB)

        def outer(t2, carry):
            for q in range(2):
                roff = rbase + (t2 * 2 + q) * GB
                pltpu.sync_copy(dst_hbm.at[pl.ds(roff, GB)], didx.at[q])
                for j in range(GB):
                    if q == 1:
                        pltpu.make_async_copy(
                            ones, acc.at[didx.at[0, j]], sem).wait()
                    else:
                        @pl.when(t2 > 0)
                        def _():
                            pltpu.make_async_copy(
                                ones, acc.at[didx.at[1, j]], sem).wait()
                    pltpu.async_copy(ones, acc.at[didx.at[q, j]], sem,
                                     add=True)
            return carry

        nw2 = NG // 2
        lax.fori_loop(0, nw2, outer, 0)
        for j in range(GB):
            pltpu.make_async_copy(ones, acc.at[didx.at[1, j]], sem).wait()
        plsc.subcore_barrier()

        def wloop(j, carry):
            pltpu.sync_copy(acc.at[pl.ds(base + j * CH, CH)], zbuf)
            pltpu.sync_copy(zbuf, out_hbm.at[c, pl.ds(base + j * CH, CH)])
            return carry

        lax.fori_loop(0, ZCH, wloop, 0)

    return pl.kernel(
        body,
        out_type=jax.ShapeDtypeStruct((NC, ACC_N, H), jnp.float32),
        mesh=_mesh,
        compiler_params=_sc_params,
        scratch_types=[
            pltpu.VMEM((CH, H), jnp.float32),
            pltpu.VMEM((2, GB, SB), jnp.int32),
            pltpu.VMEM((SB, H), jnp.float32),
            pltpu.VMEM_SHARED((ACC_N, H), jnp.float32),
            pltpu.SemaphoreType.DMA,
        ],
    )


def make_prop_kernel(g):
    CH, ZCH, RS, ACC_N, NG, EW = (
        g["CH"], g["ZCH"], g["RS"], g["ACC_N"], g["NG"], g["EW"])

    def body(src_hbm, dst_hbm, m_hbm, zeros_hbm, out_hbm,
             zbuf, sidx, didx, rows, acc, gsem, ssem):
        w, c, s = _worker_id()
        pltpu.sync_copy(zeros_hbm, zbuf)
        base = s * RS

        def zloop(j, carry):
            pltpu.sync_copy(zbuf, acc.at[pl.ds(base + j * CH, CH)])
            return carry

        lax.fori_loop(0, ZCH, zloop, 0)
        plsc.subcore_barrier()
        rbase = w * (EW // SB)

        def outer(t2, carry):
            for q in range(2):
                roff = rbase + (t2 * 2 + q) * GB
                pltpu.sync_copy(src_hbm.at[pl.ds(roff, GB)], sidx.at[q])
                pltpu.sync_copy(dst_hbm.at[pl.ds(roff, GB)], didx.at[q])
                gd = []
                for j in range(GB):
                    if q == 1:
                        pltpu.make_async_copy(
                            rows.at[j], acc.at[didx.at[0, j]], ssem).wait()
                    else:
                        @pl.when(t2 > 0)
                        def _():
                            pltpu.make_async_copy(
                                rows.at[j], acc.at[didx.at[1, j]],
                                ssem).wait()
                    gd.append(pltpu.async_copy(
                        m_hbm.at[sidx.at[q, j]], rows.at[j], gsem))
                for j in range(GB):
                    gd[j].wait()
                    pltpu.async_copy(rows.at[j], acc.at[didx.at[q, j]],
                                     ssem, add=True)
            return carry

        lax.fori_loop(0, NG // 2, outer, 0)
        for j in range(GB):
            pltpu.make_async_copy(
                rows.at[j], acc.at[didx.at[1, j]], ssem).wait()
        plsc.subcore_barrier()

        def wloop(j, carry):
            pltpu.sync_copy(acc.at[pl.ds(base + j * CH, CH)], zbuf)
            pltpu.sync_copy(zbuf, out_hbm.at[c, pl.ds(base + j * CH, CH)])
            return carry

        lax.fori_loop(0, ZCH, wloop, 0)

    return pl.kernel(
        body,
        out_type=jax.ShapeDtypeStruct((NC, ACC_N, H), jnp.float32),
        mesh=_mesh,
        compiler_params=_sc_params,
        scratch_types=[
            pltpu.VMEM((CH, H), jnp.float32),
            pltpu.VMEM((2, GB, SB), jnp.int32),
            pltpu.VMEM((2, GB, SB), jnp.int32),
            pltpu.VMEM((GB, SB, H), jnp.float32),
            pltpu.VMEM_SHARED((ACC_N, H), jnp.float32),
            pltpu.SemaphoreType.DMA,
            pltpu.SemaphoreType.DMA,
        ],
    )


def pad_edges(g, src, dst, n):
    pad = g["EPAD"] - src.shape[0]
    padi = (jnp.arange(pad, dtype=jnp.int32) % B)
    src_p = jnp.concatenate([src, padi]).reshape(g["EROWS"], SB)
    dst_p = jnp.concatenate([dst, n + padi]).reshape(g["EROWS"], SB)
    return src_p, dst_p


NB = 50
BN = N // NB
assert N % NB == 0 and BN % 8 == 0


def _prologue_body(x_ref, w0_ref, deg_ref, m_ref, dis_ref):
    deg = deg_ref[0, :, 0:1] + deg_ref[1, :, 0:1] + 1.0
    dis = lax.rsqrt(jnp.maximum(deg, 1.0))
    z = jnp.dot(x_ref[...], w0_ref[...], preferred_element_type=jnp.float32)
    dis_ref[...] = dis
    m_ref[...] = z * dis


_prologue = pl.pallas_call(
    _prologue_body,
    grid=(NB,),
    in_specs=[
        pl.BlockSpec((BN, D_IN), lambda i: (i, 0)),
        pl.BlockSpec((D_IN, H), lambda i: (0, 0)),
        pl.BlockSpec((NC, BN, H), lambda i: (0, i, 0)),
    ],
    out_specs=[
        pl.BlockSpec((BN, H), lambda i: (i, 0)),
        pl.BlockSpec((BN, 1), lambda i: (i, 0)),
    ],
    out_shape=[
        jax.ShapeDtypeStruct((N, H), jnp.float32),
        jax.ShapeDtypeStruct((N, 1), jnp.float32),
    ],
)


def _stage_body(acc_ref, m_ref, dis_ref, hmax_ref, w_ref, b_ref,
                mo_ref, ho_ref):
    dis = dis_ref[...]
    p = (acc_ref[0] + acc_ref[1] + m_ref[...]) * dis
    h = jnp.dot(p, w_ref[...], preferred_element_type=jnp.float32)
    h = jnp.maximum(h + b_ref[...], 0.0)
    ho_ref[...] = jnp.maximum(hmax_ref[...], h)
    mo_ref[...] = h * dis


_stage = pl.pallas_call(
    _stage_body,
    grid=(NB,),
    in_specs=[
        pl.BlockSpec((NC, BN, H), lambda i: (0, i, 0)),
        pl.BlockSpec((BN, H), lambda i: (i, 0)),
        pl.BlockSpec((BN, 1), lambda i: (i, 0)),
        pl.BlockSpec((BN, H), lambda i: (i, 0)),
        pl.BlockSpec((H, H), lambda i: (0, 0)),
        pl.BlockSpec((1, H), lambda i: (0, 0)),
    ],
    out_specs=[
        pl.BlockSpec((BN, H), lambda i: (i, 0)),
        pl.BlockSpec((BN, H), lambda i: (i, 0)),
    ],
    out_shape=[
        jax.ShapeDtypeStruct((N, H), jnp.float32),
        jax.ShapeDtypeStruct((N, H), jnp.float32),
    ],
)


def _final_body(acc_ref, m_ref, dis_ref, hmax_ref, w_ref, b_ref,
                fcw_ref, fcb_ref, o_ref):
    dis = dis_ref[...]
    p = (acc_ref[0] + acc_ref[1] + m_ref[...]) * dis
    h = jnp.dot(p, w_ref[...], preferred_element_type=jnp.float32)
    h = jnp.maximum(h + b_ref[...], 0.0)
    hmax = jnp.maximum(hmax_ref[...], h)
    o_ref[...] = (
        jnp.dot(hmax, fcw_ref[...], preferred_element_type=jnp.float32)
        + fcb_ref[...]
    )


_final = pl.pallas_call(
    _final_body,
    grid=(NB,),
    in_specs=[
        pl.BlockSpec((NC, BN, H), lambda i: (0, i, 0)),
        pl.BlockSpec((BN, H), lambda i: (i, 0)),
        pl.BlockSpec((BN, 1), lambda i: (i, 0)),
        pl.BlockSpec((BN, H), lambda i: (i, 0)),
        pl.BlockSpec((H, H), lambda i: (0, 0)),
        pl.BlockSpec((1, H), lambda i: (0, 0)),
        pl.BlockSpec((H, 1), lambda i: (0, 0)),
        pl.BlockSpec((1, 1), lambda i: (0, 0)),
    ],
    out_specs=pl.BlockSpec((BN, 1), lambda i: (i, 0)),
    out_shape=jax.ShapeDtypeStruct((N, 1), jnp.float32),
)


_G = _geom(N, E)
_deg_kernel = make_deg_kernel(_G)
_prop_kernel = make_prop_kernel(_G)


@jax.jit
def _run(x, edge_index, W0, b0, Ws, bs, fcW, fcb):
    src, dst = pad_edges(_G, edge_index[0], edge_index[1], N)

    zeros_prop = jnp.zeros((_G["CH"], H), jnp.float32)
    ones_deg = jnp.ones((SB, H), jnp.float32)

    degp = _deg_kernel(dst, zeros_prop, ones_deg)
    m, dis = _prologue(x, W0, degp)

    eye = jnp.eye(H, dtype=jnp.float32)
    hmax = jnp.zeros((N, H), jnp.float32)
    for k in range(L):
        Wk = eye if k == 0 else Ws[k - 1]
        bk = (b0 if k == 0 else bs[k - 1]).reshape(1, H)
        accp = _prop_kernel(src, dst, m, zeros_prop)
        if k < L - 1:
            m, hmax = _stage(accp, m, dis, hmax, Wk, bk)
        else:
            out = _final(accp, m, dis, hmax, Wk, bk,
                         fcW, fcb.reshape(1, 1))
    return out


def kernel(x, edge_index, W0, b0, Ws, bs, fcW, fcb):
    return _run(x, edge_index, W0, b0, Ws, bs, fcW, fcb)


# speedup vs baseline: 1.4335x; 1.4335x over previous
import functools

import jax
import jax.numpy as jnp
from jax import lax
from jax.experimental import pallas as pl
from jax.experimental.pallas import tpu as pltpu
from jax.experimental.pallas import tpu_sc as plsc

N = 100000
E = 3200000
D_IN = 58
H = 16
L = 10

NC = 2
NS = 16
NW = NC * NS
B = 128
SB = 256
GB = 4
GE = SB * GB


def _geom(n, e):
    g = {}
    g["EPAD"] = ((e + 2 * NW * GE - 1) // (2 * NW * GE)) * (2 * NW * GE)
    g["EW"] = g["EPAD"] // NW
    g["NG"] = g["EW"] // GE
    assert g["NG"] % 2 == 0 and g["NG"] >= 2
    g["ACC_N"] = n + B
    g["RS"] = g["ACC_N"] // NS
    assert g["RS"] * NS == g["ACC_N"]
    for zch in (42, 14, 2, 1):
        if g["RS"] % zch == 0:
            g["ZCH"] = zch
            g["CH"] = g["RS"] // zch
            break
    g["EROWS"] = g["EPAD"] // SB
    g["WROWS"] = g["EW"] // SB
    return g


_mesh = plsc.VectorSubcoreMesh(core_axis_name="c", subcore_axis_name="s")
_sc_params = pltpu.CompilerParams(use_tc_tiling_on_sc=False)


def _worker_id():
    c = lax.axis_index("c")
    s = lax.axis_index("s")
    return c * NS + s, c, s


def make_deg_kernel(g):
    CH, ZCH, RS, ACC_N, NG, EW = (
        g["CH"], g["ZCH"], g["RS"], g["ACC_N"], g["NG"], g["EW"])

    def body(dst_hbm, zeros_hbm, ones_hbm, out_hbm, zbuf, didx, ones, acc, sem):
        w, c, s = _worker_id()
        pltpu.sync_copy(ones_hbm, ones)
        pltpu.sync_copy(zeros_hbm, zbuf)
        base = s * RS

        def zloop(j, carry):
            pltpu.sync_copy(zbuf, acc.at[pl.ds(base + j * CH, CH)])
            return carry

        lax.fori_loop(0, ZCH, zloop, 0)
        plsc.subcore_barrier()
        rbase = w * (EW // SB)

        def outer(t2, carry):
            for q in range(2):
                roff = rbase + (t2 * 2 + q) * GB
                pltpu.sync_copy(dst_hbm.at[pl.ds(roff, GB)], didx.at[q])
                for j in range(GB):
                    if q == 1:
                        pltpu.make_async_copy(
                            ones, acc.at[didx.at[0, j]], sem).wait()
                    else:
                        @pl.when(t2 > 0)
                        def _():
                            pltpu.make_async_copy(
                                ones, acc.at[didx.at[1, j]], sem).wait()
                    pltpu.async_copy(ones, acc.at[didx.at[q, j]], sem,
                                     add=True)
            return carry

        nw2 = NG // 2
        lax.fori_loop(0, nw2, outer, 0)
        for j in range(GB):
            pltpu.make_async_copy(ones, acc.at[didx.at[1, j]], sem).wait()
        plsc.subcore_barrier()

        def wloop(j, carry):
            pltpu.sync_copy(acc.at[pl.ds(base + j * CH, CH)], zbuf)
            pltpu.sync_copy(zbuf, out_hbm.at[c, pl.ds(base + j * CH, CH)])
            return carry

        lax.fori_loop(0, ZCH, wloop, 0)

    return pl.kernel(
        body,
        out_type=jax.ShapeDtypeStruct((NC, ACC_N, H), jnp.float32),
        mesh=_mesh,
        compiler_params=_sc_params,
        scratch_types=[
            pltpu.VMEM((CH, H), jnp.float32),
            pltpu.VMEM((2, GB, SB), jnp.int32),
            pltpu.VMEM((SB, H), jnp.float32),
            pltpu.VMEM_SHARED((ACC_N, H), jnp.float32),
            pltpu.SemaphoreType.DMA,
        ],
    )


def make_prop_kernel(g):
    CH, ZCH, RS, ACC_N, NG, EW = (
        g["CH"], g["ZCH"], g["RS"], g["ACC_N"], g["NG"], g["EW"])

    def body(src_hbm, dst_hbm, m_hbm, zeros_hbm, out_hbm,
             zbuf, sidx, didx, rows, acc, gsem, ssem):
        w, c, s = _worker_id()
        pltpu.sync_copy(zeros_hbm, zbuf)
        base = s * RS

        def zloop(j, carry):
            pltpu.sync_copy(zbuf, acc.at[pl.ds(base + j * CH, CH)])
            return carry

        lax.fori_loop(0, ZCH, zloop, 0)
        plsc.subcore_barrier()
        rbase = w * (EW // SB)

        def outer(t2, carry):
            for q in range(2):
                roff = rbase + (t2 * 2 + q) * GB
                pltpu.sync_copy(src_hbm.at[pl.ds(roff, GB)], sidx.at[q])
                pltpu.sync_copy(dst_hbm.at[pl.ds(roff, GB)], didx.at[q])
                gd = []
                for j in range(GB):
                    if q == 1:
                        pltpu.make_async_copy(
                            rows.at[j], acc.at[didx.at[0, j]], ssem).wait()
                    else:
                        @pl.when(t2 > 0)
                        def _():
                            pltpu.make_async_copy(
                                rows.at[j], acc.at[didx.at[1, j]],
                                ssem).wait()
                    gd.append(pltpu.async_copy(
                        m_hbm.at[sidx.at[q, j]], rows.at[j], gsem))
                for j in range(GB):
                    gd[j].wait()
                    pltpu.async_copy(rows.at[j], acc.at[didx.at[q, j]],
                                     ssem, add=True)
            return carry

        lax.fori_loop(0, NG // 2, outer, 0)
        for j in range(GB):
            pltpu.make_async_copy(
                rows.at[j], acc.at[didx.at[1, j]], ssem).wait()
        plsc.subcore_barrier()

        def wloop(j, carry):
            pltpu.sync_copy(acc.at[pl.ds(base + j * CH, CH)], zbuf)
            pltpu.sync_copy(zbuf, out_hbm.at[c, pl.ds(base + j * CH, CH)])
            return carry

        lax.fori_loop(0, ZCH, wloop, 0)

    return pl.kernel(
        body,
        out_type=jax.ShapeDtypeStruct((NC, ACC_N, H), jnp.float32),
        mesh=_mesh,
        compiler_params=_sc_params,
        scratch_types=[
            pltpu.VMEM((CH, H), jnp.float32),
            pltpu.VMEM((2, GB, SB), jnp.int32),
            pltpu.VMEM((2, GB, SB), jnp.int32),
            pltpu.VMEM((GB, SB, H), jnp.float32),
            pltpu.VMEM_SHARED((ACC_N, H), jnp.float32),
            pltpu.SemaphoreType.DMA,
            pltpu.SemaphoreType.DMA,
        ],
    )


def pad_edges(g, src, dst, n):
    pad = g["EPAD"] - src.shape[0]
    padi = (jnp.arange(pad, dtype=jnp.int32) % B)
    src_p = jnp.concatenate([src, padi]).reshape(g["EROWS"], SB)
    dst_p = jnp.concatenate([dst, n + padi]).reshape(g["EROWS"], SB)
    return src_p, dst_p


BN = 2048
PBN = BN // 8
NB = -(-N // BN)
PACK_N = N * H // 128


def _prologue_body(x_ref, w0_ref, deg_ref, m_ref, dis_ref):
    deg = deg_ref[0] + deg_ref[1] + 1.0
    dis = lax.rsqrt(jnp.maximum(deg, 1.0))
    z = jnp.dot(x_ref[...], w0_ref[...], preferred_element_type=jnp.float32)
    dis_ref[...] = dis
    m_ref[...] = z * dis


def _stage_body(acc_ref, m_ref, dis_ref, hmax_ref, w_ref, b_ref,
                mo_ref, ho_ref):
    dis = dis_ref[...]
    p = (acc_ref[0] + acc_ref[1] + m_ref[...]) * dis
    h = jnp.dot(p, w_ref[...], preferred_element_type=jnp.float32)
    h = jnp.maximum(h + b_ref[...], 0.0)
    ho_ref[...] = jnp.maximum(hmax_ref[...], h)
    mo_ref[...] = h * dis


def _final_body(acc_ref, m_ref, dis_ref, hmax_ref, w_ref, b_ref,
                fcw_ref, fcb_ref, o_ref):
    dis = dis_ref[...]
    p = (acc_ref[0] + acc_ref[1] + m_ref[...]) * dis
    h = jnp.dot(p, w_ref[...], preferred_element_type=jnp.float32)
    h = jnp.maximum(h + b_ref[...], 0.0)
    hmax = jnp.maximum(hmax_ref[...], h)
    o_ref[...] = (
        jnp.dot(hmax, fcw_ref[...], preferred_element_type=jnp.float32)
        + fcb_ref[...]
    )


def _make_tc(acc_pack):
    prologue = pl.pallas_call(
        _prologue_body,
        grid=(NB,),
        in_specs=[
            pl.BlockSpec((PBN, 8 * D_IN), lambda i: (i, 0)),
            pl.BlockSpec((8 * D_IN, 128), lambda i: (0, 0)),
            pl.BlockSpec((NC, PBN, 128), lambda i: (0, i, 0)),
        ],
        out_specs=[
            pl.BlockSpec((PBN, 128), lambda i: (i, 0)),
            pl.BlockSpec((PBN, 128), lambda i: (i, 0)),
        ],
        out_shape=[
            jax.ShapeDtypeStruct((PACK_N, 128), jnp.float32),
            jax.ShapeDtypeStruct((PACK_N, 128), jnp.float32),
        ],
    )
    packed = pl.BlockSpec((PBN, 128), lambda i: (i, 0))
    accspec = pl.BlockSpec((NC, PBN, 128), lambda i: (0, i, 0))
    stage = pl.pallas_call(
        _stage_body,
        grid=(NB,),
        in_specs=[
            accspec, packed, packed, packed,
            pl.BlockSpec((128, 128), lambda i: (0, 0)),
            pl.BlockSpec((1, 128), lambda i: (0, 0)),
        ],
        out_specs=[packed, packed],
        out_shape=[
            jax.ShapeDtypeStruct((PACK_N, 128), jnp.float32),
            jax.ShapeDtypeStruct((PACK_N, 128), jnp.float32),
        ],
    )
    final = pl.pallas_call(
        _final_body,
        grid=(NB,),
        in_specs=[
            accspec, packed, packed, packed,
            pl.BlockSpec((128, 128), lambda i: (0, 0)),
            pl.BlockSpec((1, 128), lambda i: (0, 0)),
            pl.BlockSpec((128, 8), lambda i: (0, 0)),
            pl.BlockSpec((1, 8), lambda i: (0, 0)),
        ],
        out_specs=pl.BlockSpec((PBN, 8), lambda i: (i, 0)),
        out_shape=jax.ShapeDtypeStruct((PACK_N, 8), jnp.float32),
    )
    return prologue, stage, final


_G = _geom(N, E)
_deg_kernel = make_deg_kernel(_G)
_prop_kernel = make_prop_kernel(_G)
_ACC_PACK = _G["ACC_N"] * H // 128
_prologue, _stage, _final = _make_tc(_ACC_PACK)
_I8 = None


@jax.jit
def _run(x, edge_index, W0, b0, Ws, bs, fcW, fcb):
    src, dst = pad_edges(_G, edge_index[0], edge_index[1], N)

    zeros_prop = jnp.zeros((_G["CH"], H), jnp.float32)
    ones_deg = jnp.ones((SB, H), jnp.float32)

    degp = _deg_kernel(dst, zeros_prop, ones_deg)
    degr = degp.reshape(NC, _ACC_PACK, 128)
    i8 = jnp.eye(8, dtype=jnp.float32)
    xp = x.reshape(PACK_N, 8 * D_IN)
    m, dis = _prologue(xp, jnp.kron(i8, W0), degr)

    eye = jnp.eye(H, dtype=jnp.float32)
    hmax = jnp.zeros((PACK_N, 128), jnp.float32)
    for k in range(L):
        Wk = eye if k == 0 else Ws[k - 1]
        bk = b0 if k == 0 else bs[k - 1]
        W8 = jnp.kron(i8, Wk)
        b8 = jnp.tile(bk, 8).reshape(1, 128)
        accp = _prop_kernel(src, dst, m.reshape(N, H), zeros_prop)
        accr = accp.reshape(NC, _ACC_PACK, 128)
        if k < L - 1:
            m, hmax = _stage(accr, m, dis, hmax, W8, b8)
        else:
            outp = _final(accr, m, dis, hmax, W8, b8,
                          jnp.kron(i8, fcW), jnp.tile(fcb, 8).reshape(1, 8))
    return outp.reshape(N, 1)


def kernel(x, edge_index, W0, b0, Ws, bs, fcW, fcb):
    return _run(x, edge_index, W0, b0, Ws, bs, fcW, fcb)


# speedup vs baseline: 1.4342x; 1.0005x over previous
import jax
import jax.numpy as jnp
from jax import lax
from jax.experimental import pallas as pl
from jax.experimental.pallas import tpu as pltpu
from jax.experimental.pallas import tpu_sc as plsc

N = 100000
E = 3200000
D_IN = 58
H = 16
L = 10

NC = 2
NS = 16
NW = NC * NS
B = 128
SB = 256
GB = 4
GE = SB * GB


def _geom(n, e):
    g = {}
    g["EPAD"] = ((e + 2 * NW * GE - 1) // (2 * NW * GE)) * (2 * NW * GE)
    g["EW"] = g["EPAD"] // NW
    g["NG"] = g["EW"] // GE
    assert g["NG"] % 2 == 0 and g["NG"] >= 2
    g["ACC_N"] = n + B
    g["RS"] = g["ACC_N"] // NS
    assert g["RS"] * NS == g["ACC_N"]
    for zch in (42, 14, 2, 1):
        if g["RS"] % zch == 0:
            g["ZCH"] = zch
            g["CH"] = g["RS"] // zch
            break
    g["EROWS"] = g["EPAD"] // SB
    g["WROWS"] = g["EW"] // SB
    return g


_mesh = plsc.VectorSubcoreMesh(core_axis_name="c", subcore_axis_name="s")
_sc_params = pltpu.CompilerParams(use_tc_tiling_on_sc=False)


def _worker_id():
    c = lax.axis_index("c")
    s = lax.axis_index("s")
    return c * NS + s, c, s


def make_deg_kernel(g):
    CH, ZCH, RS, ACC_N, NG, EW = (
        g["CH"], g["ZCH"], g["RS"], g["ACC_N"], g["NG"], g["EW"])

    def body(dst_hbm, zeros_hbm, ones_hbm, out_hbm, zbuf, didx, ones, acc, sem):
        w, c, s = _worker_id()
        pltpu.sync_copy(ones_hbm, ones)
        pltpu.sync_copy(zeros_hbm, zbuf)
        base = s * RS

        def zloop(j, carry):
            pltpu.sync_copy(zbuf, acc.at[pl.ds(base + j * CH, CH)])
            return carry

        lax.fori_loop(0, ZCH, zloop, 0)
        plsc.subcore_barrier()
        rbase = w * (EW // SB)

        def outer(t2, carry):
            for q in range(2):
                roff = rbase + (t2 * 2 + q) * GB
                pltpu.sync_copy(dst_hbm.at[pl.ds(roff, GB)], didx.at[q])
                for j in range(GB):
                    if q == 1:
                        pltpu.make_async_copy(
                            ones, acc.at[didx.at[0, j]], sem).wait()
                    else:
                        @pl.when(t2 > 0)
                        def _():
                            pltpu.make_async_copy(
                                ones, acc.at[didx.at[1, j]], sem).wait()
                    pltpu.async_copy(ones, acc.at[didx.at[q, j]], sem,
                                     add=True)
            return carry

        nw2 = NG // 2
        lax.fori_loop(0, nw2, outer, 0)
        for j in range(GB):
            pltpu.make_async_copy(ones, acc.at[didx.at[1, j]], sem).wait()
        plsc.subcore_barrier()

        def wloop(j, carry):
            pltpu.sync_copy(acc.at[pl.ds(base + j * CH, CH)], zbuf)
            pltpu.sync_copy(zbuf, out_hbm.at[c, pl.ds(base + j * CH, CH)])
            return carry

        lax.fori_loop(0, ZCH, wloop, 0)

    return pl.kernel(
        body,
        out_type=jax.ShapeDtypeStruct((NC, ACC_N, H), jnp.float32),
        mesh=_mesh,
        compiler_params=_sc_params,
        scratch_types=[
            pltpu.VMEM((CH, H), jnp.float32),
            pltpu.VMEM((2, GB, SB), jnp.int32),
            pltpu.VMEM((SB, H), jnp.float32),
            pltpu.VMEM_SHARED((ACC_N, H), jnp.float32),
            pltpu.SemaphoreType.DMA,
        ],
    )


def make_prop_kernel(g):
    CH, ZCH, RS, ACC_N, NG, EW = (
        g["CH"], g["ZCH"], g["RS"], g["ACC_N"], g["NG"], g["EW"])

    def body(src_hbm, dst_hbm, m_hbm, zeros_hbm, out_hbm,
             zbuf, sidx, didx, rows, acc, gsem, ssem):
        w, c, s = _worker_id()
        pltpu.sync_copy(zeros_hbm, zbuf)
        base = s * RS

        def zloop(j, carry):
            pltpu.sync_copy(zbuf, acc.at[pl.ds(base + j * CH, CH)])
            return carry

        lax.fori_loop(0, ZCH, zloop, 0)
        plsc.subcore_barrier()
        rbase = w * (EW // SB)

        def outer(t2, carry):
            for q in range(2):
                roff = rbase + (t2 * 2 + q) * GB
                pltpu.sync_copy(src_hbm.at[pl.ds(roff, GB)], sidx.at[q])
                pltpu.sync_copy(dst_hbm.at[pl.ds(roff, GB)], didx.at[q])
                gd = []
                for j in range(GB):
                    if q == 1:
                        pltpu.make_async_copy(
                            rows.at[j], acc.at[didx.at[0, j]], ssem).wait()
                    else:
                        @pl.when(t2 > 0)
                        def _():
                            pltpu.make_async_copy(
                                rows.at[j], acc.at[didx.at[1, j]],
                                ssem).wait()
                    gd.append(pltpu.async_copy(
                        m_hbm.at[sidx.at[q, j]], rows.at[j], gsem))
                for j in range(GB):
                    gd[j].wait()
                    pltpu.async_copy(rows.at[j], acc.at[didx.at[q, j]],
                                     ssem, add=True)
            return carry

        lax.fori_loop(0, NG // 2, outer, 0)
        for j in range(GB):
            pltpu.make_async_copy(
                rows.at[j], acc.at[didx.at[1, j]], ssem).wait()
        plsc.subcore_barrier()

        def wloop(j, carry):
            pltpu.sync_copy(acc.at[pl.ds(base + j * CH, CH)], zbuf)
            pltpu.sync_copy(zbuf, out_hbm.at[c, pl.ds(base + j * CH, CH)])
            return carry

        lax.fori_loop(0, ZCH, wloop, 0)

    return pl.kernel(
        body,
        out_type=jax.ShapeDtypeStruct((NC, ACC_N, H), jnp.float32),
        mesh=_mesh,
        compiler_params=_sc_params,
        scratch_types=[
            pltpu.VMEM((CH, H), jnp.float32),
            pltpu.VMEM((2, GB, SB), jnp.int32),
            pltpu.VMEM((2, GB, SB), jnp.int32),
            pltpu.VMEM((GB, SB, H), jnp.float32),
            pltpu.VMEM_SHARED((ACC_N, H), jnp.float32),
            pltpu.SemaphoreType.DMA,
            pltpu.SemaphoreType.DMA,
        ],
    )


def pad_edges(g, src, dst, n):
    pad = g["EPAD"] - src.shape[0]
    padi = (jnp.arange(pad, dtype=jnp.int32) % B)
    src_p = jnp.concatenate([src, padi]).reshape(g["EROWS"], SB)
    dst_p = jnp.concatenate([dst, n + padi]).reshape(g["EROWS"], SB)
    return src_p, dst_p


BN = 2048
PBN = BN // 8
NB = -(-N // BN)
PACK_N = N * H // 128


def _prologue_body(x_ref, w0_ref, deg_ref, m_ref, dis_ref):
    deg = deg_ref[0] + deg_ref[1] + 1.0
    dis = lax.rsqrt(jnp.maximum(deg, 1.0))
    z = jnp.dot(x_ref[...], w0_ref[...], preferred_element_type=jnp.float32)
    dis_ref[...] = dis
    m_ref[...] = z * dis


def _stage_body(acc_ref, m_ref, dis_ref, hmax_ref, w_ref, b_ref,
                mo_ref, ho_ref):
    dis = dis_ref[...]
    p = (acc_ref[0] + acc_ref[1] + m_ref[...]) * dis
    h = jnp.dot(p, w_ref[...], preferred_element_type=jnp.float32)
    h = jnp.maximum(h + b_ref[...], 0.0)
    ho_ref[...] = jnp.maximum(hmax_ref[...], h)
    mo_ref[...] = h * dis


def _final_body(acc_ref, m_ref, dis_ref, hmax_ref, w_ref, b_ref,
                fcw_ref, fcb_ref, o_ref):
    dis = dis_ref[...]
    p = (acc_ref[0] + acc_ref[1] + m_ref[...]) * dis
    h = jnp.dot(p, w_ref[...], preferred_element_type=jnp.float32)
    h = jnp.maximum(h + b_ref[...], 0.0)
    hmax = jnp.maximum(hmax_ref[...], h)
    o_ref[...] = (
        jnp.dot(hmax, fcw_ref[...], preferred_element_type=jnp.float32)
        + fcb_ref[...]
    )


def _make_tc(acc_pack):
    prologue = pl.pallas_call(
        _prologue_body,
        grid=(NB,),
        in_specs=[
            pl.BlockSpec((PBN, 8 * D_IN), lambda i: (i, 0)),
            pl.BlockSpec((8 * D_IN, 128), lambda i: (0, 0)),
            pl.BlockSpec((NC, PBN, 128), lambda i: (0, i, 0)),
        ],
        out_specs=[
            pl.BlockSpec((PBN, 128), lambda i: (i, 0)),
            pl.BlockSpec((PBN, 128), lambda i: (i, 0)),
        ],
        out_shape=[
            jax.ShapeDtypeStruct((PACK_N, 128), jnp.float32),
            jax.ShapeDtypeStruct((PACK_N, 128), jnp.float32),
        ],
    )
    packed = pl.BlockSpec((PBN, 128), lambda i: (i, 0))
    accspec = pl.BlockSpec((NC, PBN, 128), lambda i: (0, i, 0))
    stage = pl.pallas_call(
        _stage_body,
        grid=(NB,),
        in_specs=[
            accspec, packed, packed, packed,
            pl.BlockSpec((128, 128), lambda i: (0, 0)),
            pl.BlockSpec((1, 128), lambda i: (0, 0)),
        ],
        out_specs=[packed, packed],
        out_shape=[
            jax.ShapeDtypeStruct((PACK_N, 128), jnp.float32),
            jax.ShapeDtypeStruct((PACK_N, 128), jnp.float32),
        ],
    )
    final = pl.pallas_call(
        _final_body,
        grid=(NB,),
        in_specs=[
            accspec, packed, packed, packed,
            pl.BlockSpec((128, 128), lambda i: (0, 0)),
            pl.BlockSpec((1, 128), lambda i: (0, 0)),
            pl.BlockSpec((128, 8), lambda i: (0, 0)),
            pl.BlockSpec((1, 8), lambda i: (0, 0)),
        ],
        out_specs=pl.BlockSpec((PBN, 8), lambda i: (i, 0)),
        out_shape=jax.ShapeDtypeStruct((PACK_N, 8), jnp.float32),
    )
    return prologue, stage, final


_G = _geom(N, E)
_deg_kernel = make_deg_kernel(_G)
_prop_kernel = make_prop_kernel(_G)
_ACC_PACK = _G["ACC_N"] * H // 128
_prologue, _stage, _final = _make_tc(_ACC_PACK)


@jax.jit
def _run(x, edge_index, W0, b0, Ws, bs, fcW, fcb):
    src, dst = pad_edges(_G, edge_index[0], edge_index[1], N)

    zeros_prop = jnp.zeros((_G["CH"], H), jnp.float32)
    ones_deg = jnp.ones((SB, H), jnp.float32)

    degp = _deg_kernel(dst, zeros_prop, ones_deg)
    degr = degp.reshape(NC, _ACC_PACK, 128)
    i8 = jnp.eye(8, dtype=jnp.float32)
    xp = x.reshape(PACK_N, 8 * D_IN)
    m, dis = _prologue(xp, jnp.kron(i8, W0), degr)

    eye = jnp.eye(H, dtype=jnp.float32)
    hmax = jnp.zeros((PACK_N, 128), jnp.float32)
    for k in range(L):
        Wk = eye if k == 0 else Ws[k - 1]
        bk = b0 if k == 0 else bs[k - 1]
        W8 = jnp.kron(i8, Wk)
        b8 = jnp.tile(bk, 8).reshape(1, 128)
        accp = _prop_kernel(src, dst, m.reshape(N, H), zeros_prop)
        accr = accp.reshape(NC, _ACC_PACK, 128)
        if k < L - 1:
            m, hmax = _stage(accr, m, dis, hmax, W8, b8)
        else:
            outp = _final(accr, m, dis, hmax, W8, b8,
                          jnp.kron(i8, fcW), jnp.tile(fcb, 8).reshape(1, 8))
    return outp.reshape(N, 1)


def kernel(x, edge_index, W0, b0, Ws, bs, fcW, fcb):
    return _run(x, edge_index, W0, b0, Ws, bs, fcW, fcb)
